# Initial kernel scaffold; baseline (speedup 1.0000x reference)
import jax
import jax.numpy as jnp
from jax.experimental import pallas as pl


def kernel(node_repr, rel_emb, query_src_ts_emb, query_rel_emb, visited_node_score, Wq, Wk, W_lin, b_lin, edge_src, edge_dst, query_idx):
    raise NotImplementedError("write your pallas kernel here")


# speedup vs baseline: 1.1136x; 1.1136x over previous
import jax
import jax.numpy as jnp
from jax import lax
from jax.experimental import pallas as pl
from jax.experimental.pallas import tpu as pltpu
from jax.experimental.pallas import tpu_sc as plsc

N = 10000
E = 160000
D = 128
B = 128
NEG_SLOPE = 0.01

NPAD = 10240
EPAD = 163840
NC = 2
NS = 16
NW = NC * NS
EW = EPAD // NW
C1 = 32
C3 = 64
C4 = 32
QCOLS = 256


def _hdot(a, b):
  return jax.lax.dot_general(a, b, (((a.ndim - 1,), (0,)), ((), ())),
                             precision=jax.lax.Precision.HIGHEST,
                             preferred_element_type=jnp.float32)

def _t0_body(wq_ref, wk_ref, qst_ref, qr_ref, m_ref, qtab_ref, u_ref, v_ref):
  wq = wq_ref[...]
  wk = wk_ref[...]
  m = jax.lax.dot_general(wq, wk, (((0,), (0,)), ((), ())),
                          precision=jax.lax.Precision.HIGHEST,
                          preferred_element_type=jnp.float32)
  m_ref[...] = m

  qst = qst_ref[...]
  qr = qr_ref[...]

  def blk(a, b):
    return m[a * D:(a + 1) * D, b * D:(b + 1) * D]

  h2 = (_hdot(qst, blk(1, 2).T) + _hdot(qr, blk(1, 3).T)
        + _hdot(qst, blk(2, 1)) + _hdot(qr, blk(3, 1)))
  c = (jnp.sum(qst * (_hdot(qst, blk(2, 2).T) + _hdot(qr, blk(2, 3).T)), axis=1)
       + jnp.sum(qr * (_hdot(qst, blk(3, 2).T) + _hdot(qr, blk(3, 3).T)), axis=1))
  u = _hdot(qst, blk(0, 2).T) + _hdot(qr, blk(0, 3).T)
  v = _hdot(qst, blk(2, 0)) + _hdot(qr, blk(3, 0))
  qtab_ref[...] = jnp.concatenate(
      [h2, c[:, None], jnp.zeros((B, QCOLS - D - 1), jnp.float32)], axis=1)
  u_ref[...] = u
  v_ref[...] = v


def _tables_small(Wq, Wk, qst, qr):
  return pl.pallas_call(
      _t0_body,
      out_shape=(
          jax.ShapeDtypeStruct((4 * D, 4 * D), jnp.float32),
          jax.ShapeDtypeStruct((B, QCOLS), jnp.float32),
          jax.ShapeDtypeStruct((B, D), jnp.float32),
          jax.ShapeDtypeStruct((B, D), jnp.float32),
      ),
  )(Wq, Wk, qst, qr)


def _t1_body(nb_ref, m_ref, u_ref, v_ref, tsrc_ref, tdst_ref):
  nb = nb_ref[...]
  m = m_ref[...]

  def blk(a, b):
    return m[a * D:(a + 1) * D, b * D:(b + 1) * D]

  gp = _hdot(nb, blk(0, 1))
  p = _hdot(nb, blk(0, 0))
  x = _hdot(nb, u_ref[...].T)
  g = _hdot(nb, blk(1, 0).T)
  y = _hdot(nb, v_ref[...].T)
  tsrc_ref[...] = jnp.concatenate([gp, p, x], axis=1)
  tdst_ref[...] = jnp.concatenate([g, nb, y], axis=1)


def _tables_node(node_pad, M, U, V):
  grid = NPAD // 512
  return pl.pallas_call(
      _t1_body,
      grid=(grid,),
      in_specs=[
          pl.BlockSpec((512, D), lambda i: (i, 0)),
          pl.BlockSpec((4 * D, 4 * D), lambda i: (0, 0)),
          pl.BlockSpec((B, D), lambda i: (0, 0)),
          pl.BlockSpec((B, D), lambda i: (0, 0)),
      ],
      out_specs=(
          pl.BlockSpec((512, 3 * D), lambda i: (i, 0)),
          pl.BlockSpec((512, 3 * D), lambda i: (i, 0)),
      ),
      out_shape=(
          jax.ShapeDtypeStruct((NPAD, 3 * D), jnp.float32),
          jax.ShapeDtypeStruct((NPAD, 3 * D), jnp.float32),
      ),
  )(node_pad, M, U, V)


def _t2_body(rel_ref, m11_ref, z_ref):
  relb = rel_ref[...]
  m11 = m11_ref[...]
  t = jax.lax.dot_general(relb, m11, (((2,), (1,)), ((), ())),
                          precision=jax.lax.Precision.HIGHEST,
                          preferred_element_type=jnp.float32)
  z_ref[...] = jnp.sum(relb * t, axis=2)


def _quad_form(rel3, M11):
  grid = 25
  return pl.pallas_call(
      _t2_body,
      grid=(grid,),
      in_specs=[
          pl.BlockSpec((40, 160, D), lambda i: (i, 0, 0)),
          pl.BlockSpec((D, D), lambda i: (0, 0)),
      ],
      out_specs=pl.BlockSpec((40, 160), lambda i: (i, 0)),
      out_shape=jax.ShapeDtypeStruct((1000, 160), jnp.float32),
  )(rel3, M11)


def _t5_body(ag_ref, sc_ref, ms_ref, nb_ref, w_ref, b_ref, out_ref, score_ref):
  agg = ag_ref[0] + ag_ref[1]
  mask = ms_ref[...]
  upd = agg + mask * nb_ref[...]
  uu = jax.lax.bitcast_convert_type(upd, jnp.int32)
  ur = (uu + 0x7FFF + ((uu >> 16) & 1)) & jnp.int32(-65536)
  updr = jax.lax.bitcast_convert_type(ur, jnp.float32)
  out = _hdot(updr, w_ref[...].T) + b_ref[...]
  out_ref[...] = jnp.where(out >= 0.0, out, NEG_SLOPE * out)
  score_ref[...] = sc_ref[0] + sc_ref[1]


def _finalize(agg2, score2r, ms2d, node_pad, W_lin, b_lin2):
  grid = NPAD // 512
  return pl.pallas_call(
      _t5_body,
      grid=(grid,),
      in_specs=[
          pl.BlockSpec((2, 512, D), lambda i: (0, i, 0)),
          pl.BlockSpec((2, 1, 1, 512), lambda i: (0, i, 0, 0)),
          pl.BlockSpec((512, 1), lambda i: (i, 0)),
          pl.BlockSpec((512, D), lambda i: (i, 0)),
          pl.BlockSpec((D, D), lambda i: (0, 0)),
          pl.BlockSpec((1, D), lambda i: (0, 0)),
      ],
      out_specs=(
          pl.BlockSpec((512, D), lambda i: (i, 0)),
          pl.BlockSpec((1, 1, 512), lambda i: (i, 0, 0)),
      ),
      out_shape=(
          jax.ShapeDtypeStruct((NPAD, D), jnp.float32),
          jax.ShapeDtypeStruct((NPAD // 512, 1, 512), jnp.float32),
      ),
  )(agg2, score2r, ms2d, node_pad, W_lin, b_lin2)


_MESH = plsc.VectorSubcoreMesh(core_axis_name="c", subcore_axis_name="s")


def _wid():
  return lax.axis_index("s") * NC + lax.axis_index("c")


def _iota16():
  return lax.iota(jnp.int32, 16)


def _splat(x):
  return jnp.broadcast_to(x, (16,))


def _p1_body(tsrc, tdst, qtab, relh, esh, edh, qih, zh,
             logits_out, lmax_out,
             bufS, bufD, bufQ, relbuf, isrcv, idstv, iqv, eidv,
             zbuf, lgbuf, kscr, vscr, lmax):
  wid = _wid()
  ebase = wid * EW

  neg = jnp.full((16,), -1e30, jnp.float32)

  def init_body(k, _):
    lmax[pl.ds(k * 16, 16)] = neg
    return 0

  lax.fori_loop(0, NPAD // 16, init_body, 0)

  pltpu.sync_copy(qtab, bufQ)

  iota = _iota16()
  emax = _splat(E - 1)

  def round_body(r, _):
    base = ebase + r * C1
    pltpu.sync_copy(esh.at[pl.ds(base, C1)], isrcv)
    pltpu.sync_copy(edh.at[pl.ds(base, C1)], idstv)
    pltpu.sync_copy(qih.at[pl.ds(base, C1)], iqv)
    pltpu.sync_copy(zh.at[pl.ds(base, C1)], zbuf)
    for g in range(C1 // 16):
      eidv[pl.ds(g * 16, 16)] = jnp.minimum(_splat(base + g * 16) + iota, emax)
    pltpu.sync_copy(tsrc.at[isrcv], bufS)
    pltpu.sync_copy(tdst.at[idstv], bufD)
    pltpu.sync_copy(relh.at[eidv], relbuf)

    for g in range(C1 // 16):
      lane = iota + (g * 16)
      s16 = isrcv[pl.ds(g * 16, 16)]
      iq16 = iqv[pl.ds(g * 16, 16)]
      z16 = zbuf[pl.ds(g * 16, 16)]
      c16 = plsc.load_gather(bufQ, [iq16, _splat(D)])
      x16 = plsc.load_gather(bufS, [lane, _splat(2 * D) + iq16])
      y16 = plsc.load_gather(bufD, [lane, _splat(2 * D) + iq16])
      acc0 = z16 + c16 + x16 + y16

      def dot_body(j, acc):
        jb = _splat(j)
        r16 = plsc.load_gather(relbuf, [lane, jb])
        gp16 = plsc.load_gather(bufS, [lane, jb])
        p16 = plsc.load_gather(bufS, [lane, _splat(D) + jb])
        g16 = plsc.load_gather(bufD, [lane, jb])
        nr16 = plsc.load_gather(bufD, [lane, _splat(D) + jb])
        h16 = plsc.load_gather(bufQ, [iq16, jb])
        return acc + r16 * (gp16 + g16 + h16) + p16 * nr16

      acc = lax.fori_loop(0, D, dot_body, acc0)
      lgbuf[pl.ds(g * 16, 16)] = acc

      key, val = plsc.sort_key_val(s16, acc)
      for sh in (1, 2, 4, 8):
        kscr[...] = key
        vscr[...] = val
        back = jnp.maximum(iota - sh, 0)
        kb = plsc.load_gather(kscr, [back])
        vb = plsc.load_gather(vscr, [back])
        same = jnp.logical_and(kb == key, iota >= sh)
        val = jnp.where(same, jnp.maximum(val, vb), val)
      kscr[...] = key
      nxt = jnp.minimum(iota + 1, 15)
      kn = plsc.load_gather(kscr, [nxt])
      is_last = jnp.logical_or(kn != key, iota == 15)
      cur = plsc.load_gather(lmax, [key])
      plsc.store_scatter(lmax, [key], jnp.maximum(cur, val), mask=is_last)

    pltpu.sync_copy(lgbuf, logits_out.at[pl.ds(base, C1)])
    return 0

  lax.fori_loop(0, EW // C1, round_body, 0)
  pltpu.sync_copy(lmax, lmax_out.at[wid])


def _run_p1(tsrc, tdst, qtab, relh, esh, edh, qih, zh):
  return pl.kernel(
      _p1_body,
      out_type=(
          jax.ShapeDtypeStruct((EPAD,), jnp.float32),
          jax.ShapeDtypeStruct((NW, NPAD), jnp.float32),
      ),
      mesh=_MESH,
      compiler_params=pltpu.CompilerParams(use_tc_tiling_on_sc=False, needs_layout_passes=False),
      scratch_types=[
          pltpu.VMEM((C1, 3 * D), jnp.float32),
          pltpu.VMEM((C1, 3 * D), jnp.float32),
          pltpu.VMEM((B, QCOLS), jnp.float32),
          pltpu.VMEM((C1, D), jnp.float32),
          pltpu.VMEM((C1,), jnp.int32),
          pltpu.VMEM((C1,), jnp.int32),
          pltpu.VMEM((C1,), jnp.int32),
          pltpu.VMEM((C1,), jnp.int32),
          pltpu.VMEM((C1,), jnp.float32),
          pltpu.VMEM((C1,), jnp.float32),
          pltpu.VMEM((16,), jnp.int32),
          pltpu.VMEM((16,), jnp.float32),
          pltpu.VMEM((NPAD,), jnp.float32),
      ],
  )(tsrc, tdst, qtab, relh, esh, edh, qih, zh)


def _p3_body(lmaxh, logitsh, esh, exh, denomh,
             gmax, mbuf, lgbuf, exbuf, isrcv, zv, denom_sp):
  cid = lax.axis_index("c")
  sid = lax.axis_index("s")
  wid = sid * NC + cid
  ebase = wid * EW

  zero = jnp.zeros((16,), jnp.float32)

  def z_body(k, _):
    zv[pl.ds(k * 16, 16)] = zero
    return 0

  lax.fori_loop(0, (NPAD // NS) // 16, z_body, 0)
  pltpu.sync_copy(zv, denom_sp.at[pl.ds(sid * (NPAD // NS), NPAD // NS)])

  def cmb_outer(kk, _):
    pltpu.sync_copy(lmaxh.at[:, pl.ds(kk * 2048, 2048)], mbuf)

    def cmb_inner(j, _):
      m = mbuf[0, pl.ds(j * 16, 16)]
      for w in range(1, NW):
        m = jnp.maximum(m, mbuf[w, pl.ds(j * 16, 16)])
      gmax[pl.ds(kk * 2048 + j * 16, 16)] = m
      return 0

    lax.fori_loop(0, 2048 // 16, cmb_inner, 0)
    return 0

  lax.fori_loop(0, NPAD // 2048, cmb_outer, 0)
  plsc.subcore_barrier()

  def round_body(r, _):
    base = ebase + r * C3
    pltpu.sync_copy(logitsh.at[pl.ds(base, C3)], lgbuf)
    pltpu.sync_copy(esh.at[pl.ds(base, C3)], isrcv)
    for g in range(C3 // 16):
      lg16 = lgbuf[pl.ds(g * 16, 16)]
      s16 = isrcv[pl.ds(g * 16, 16)]
      gm16 = plsc.load_gather(gmax, [s16])
      exbuf[pl.ds(g * 16, 16)] = jnp.exp(lg16 - gm16)
    pltpu.sync_copy(exbuf, exh.at[pl.ds(base, C3)])
    pltpu.sync_copy(exbuf, denom_sp.at[isrcv], add=True)
    return 0

  lax.fori_loop(0, EW // C3, round_body, 0)
  plsc.subcore_barrier()
  sl = pl.ds(sid * (NPAD // NS), NPAD // NS)
  pltpu.sync_copy(denom_sp.at[sl], denomh.at[cid, sl])


def _run_p3(lmaxh, logitsh, esh):
  return pl.kernel(
      _p3_body,
      out_type=(
          jax.ShapeDtypeStruct((EPAD,), jnp.float32),
          jax.ShapeDtypeStruct((NC, NPAD), jnp.float32),
      ),
      mesh=_MESH,
      compiler_params=pltpu.CompilerParams(use_tc_tiling_on_sc=False, needs_layout_passes=False),
      scratch_types=[
          pltpu.VMEM((NPAD,), jnp.float32),
          pltpu.VMEM((NW, 2048), jnp.float32),
          pltpu.VMEM((C3,), jnp.float32),
          pltpu.VMEM((C3,), jnp.float32),
          pltpu.VMEM((C3,), jnp.int32),
          pltpu.VMEM((NPAD // NS,), jnp.float32),
          pltpu.VMEM_SHARED((NPAD,), jnp.float32),
      ],
  )(lmaxh, logitsh, esh)


def _p4_body(exh, esh, edh, denomh, vnsh, nodeh,
             scoreh, aggh, msh,
             denv, vnsv, dbuf, exbuf, smbuf, sbuf, isrcv, idstv,
             ndbuf, zrows, msv, score_sp, agg_sp, semN):
  cid = lax.axis_index("c")
  sid = lax.axis_index("s")
  wid = sid * NC + cid
  ebase = wid * EW
  rows = NPAD // NS

  zero = jnp.zeros((16,), jnp.float32)

  def zr_body(k, _):
    for jj in range(D // 16):
      zrows[k, pl.ds(jj * 16, 16)] = zero
    return 0

  lax.fori_loop(0, 40, zr_body, 0)

  def zv_body(k, _):
    msv[pl.ds(k * 16, 16)] = zero
    return 0

  lax.fori_loop(0, rows // 16, zv_body, 0)
  pltpu.sync_copy(msv, score_sp.at[pl.ds(sid * rows, rows)])
  for t in range(16):
    pltpu.sync_copy(zrows, agg_sp.at[pl.ds(sid * rows + t * 40, 40), :])

  def den_outer(kk, _):
    pltpu.sync_copy(denomh.at[:, pl.ds(kk * 2048, 2048)], dbuf)

    def den_inner(j, _):
      denv[pl.ds(kk * 2048 + j * 16, 16)] = (
          dbuf[0, pl.ds(j * 16, 16)] + dbuf[1, pl.ds(j * 16, 16)])
      return 0

    lax.fori_loop(0, 2048 // 16, den_inner, 0)
    return 0

  lax.fori_loop(0, NPAD // 2048, den_outer, 0)
  pltpu.sync_copy(vnsh, vnsv)
  plsc.subcore_barrier()

  iota = _iota16()

  def round_body(r, _):
    base = ebase + r * C4
    pltpu.sync_copy(exh.at[pl.ds(base, C4)], exbuf)
    pltpu.sync_copy(esh.at[pl.ds(base, C4)], isrcv)
    pltpu.sync_copy(edh.at[pl.ds(base, C4)], idstv)
    dN = pltpu.async_copy(nodeh.at[idstv], ndbuf, semN)
    sms = []
    for g in range(C4 // 16):
      e16 = exbuf[pl.ds(g * 16, 16)]
      s16 = isrcv[pl.ds(g * 16, 16)]
      den16 = plsc.load_gather(denv, [s16])
      sm16 = e16 / (den16 + 1e-16)
      vn16 = plsc.load_gather(vnsv, [s16])
      sbuf[pl.ds(g * 16, 16)] = sm16 * vn16
      smbuf[pl.ds(g * 16, 16)] = sm16
      sms.append(sm16)
    dN.wait()

    def sc_body(j, _):
      jb = _splat(j)
      for g in range(C4 // 16):
        lane = iota + (g * 16)
        val = plsc.load_gather(ndbuf, [lane, jb]) * sms[g]
        plsc.store_scatter(ndbuf, [lane, jb], val)
      return 0

    lax.fori_loop(0, D, sc_body, 0)
    pltpu.sync_copy(ndbuf, agg_sp.at[isrcv], add=True)
    pltpu.sync_copy(sbuf, score_sp.at[idstv], add=True)
    return 0

  lax.fori_loop(0, EW // C4, round_body, 0)

  def ms_body(k, _):
    d16 = denv[pl.ds(sid * rows + k * 16, 16)]
    msv[pl.ds(k * 16, 16)] = jnp.where(d16 > 0.0, 0.0, 1.0)
    return 0

  lax.fori_loop(0, rows // 16, ms_body, 0)

  plsc.subcore_barrier()
  sl = pl.ds(sid * rows, rows)
  pltpu.sync_copy(score_sp.at[sl], scoreh.at[cid, sl])
  pltpu.sync_copy(agg_sp.at[sl, :], aggh.at[cid, sl, :])

  @pl.when(cid == 0)
  def _():
    pltpu.sync_copy(msv, msh.at[sl])


def _run_p4(exh, esh, edh, denomh, vnsh, nodeh):
  return pl.kernel(
      _p4_body,
      out_type=(
          jax.ShapeDtypeStruct((NC, NPAD), jnp.float32),
          jax.ShapeDtypeStruct((NC, NPAD, D), jnp.float32),
          jax.ShapeDtypeStruct((NPAD,), jnp.float32),
      ),
      mesh=_MESH,
      compiler_params=pltpu.CompilerParams(use_tc_tiling_on_sc=False, needs_layout_passes=False),
      scratch_types=[
          pltpu.VMEM((NPAD,), jnp.float32),
          pltpu.VMEM((NPAD,), jnp.float32),
          pltpu.VMEM((NC, 2048), jnp.float32),
          pltpu.VMEM((C4,), jnp.float32),
          pltpu.VMEM((C4,), jnp.float32),
          pltpu.VMEM((C4,), jnp.float32),
          pltpu.VMEM((C4,), jnp.int32),
          pltpu.VMEM((C4,), jnp.int32),
          pltpu.VMEM((C4, D), jnp.float32),
          pltpu.VMEM((40, D), jnp.float32),
          pltpu.VMEM((NPAD // NS,), jnp.float32),
          pltpu.VMEM_SHARED((NPAD,), jnp.float32),
          pltpu.VMEM_SHARED((NPAD, D), jnp.float32),
          pltpu.SemaphoreType.DMA,
      ],
  )(exh, esh, edh, denomh, vnsh, nodeh)


@jax.jit
def kernel(node_repr, rel_emb, query_src_ts_emb, query_rel_emb,
           visited_node_score, Wq, Wk, W_lin, b_lin,
           edge_src, edge_dst, query_idx):
  def _r(x):
    u = jax.lax.bitcast_convert_type(x, jnp.int32)
    r = (u + 0x7FFF + ((u >> 16) & 1)) & jnp.int32(-65536)
    return jax.lax.bitcast_convert_type(r, jnp.float32)

  node_pad = jnp.concatenate(
      [node_repr, jnp.zeros((NPAD - N, D), jnp.float32)], axis=0)
  node_pad_r = _r(node_pad)
  rel_r = _r(rel_emb)
  vns_pad = jnp.concatenate(
      [visited_node_score, jnp.zeros((NPAD - N,), jnp.float32)])
  pad_i = jnp.full((EPAD - E,), N, jnp.int32)
  es_pad = jnp.concatenate([edge_src, pad_i])
  ed_pad = jnp.concatenate([edge_dst, pad_i])
  qi_pad = jnp.concatenate([query_idx, jnp.zeros((EPAD - E,), jnp.int32)])

  M, qtab, U, V = _tables_small(_r(Wq), _r(Wk), _r(query_src_ts_emb),
                                _r(query_rel_emb))
  tsrc, tdst = _tables_node(node_pad_r, M, U, V)
  z2d = _quad_form(rel_r.reshape(1000, 160, D), M[D:2 * D, D:2 * D])
  z_pad = jnp.concatenate(
      [z2d.reshape(E), jnp.zeros((EPAD - E,), jnp.float32)])

  logits, lmaxh = _run_p1(tsrc, tdst, qtab, rel_r,
                          es_pad, ed_pad, qi_pad, z_pad)
  exh, denomh = _run_p3(lmaxh, logits, es_pad)
  scoreh, aggh, msh = _run_p4(exh, es_pad, ed_pad, denomh, vns_pad, node_pad)

  out_repr_pad, score2d = _finalize(
      aggh, scoreh.reshape(NC, NPAD // 512, 1, 512), msh.reshape(NPAD, 1),
      node_pad, _r(W_lin), b_lin.reshape(1, D))
  return score2d.reshape(NPAD)[:N], out_repr_pad[:N]


# speedup vs baseline: 1.2407x; 1.1141x over previous
import jax
import jax.numpy as jnp
from jax import lax
from jax.experimental import pallas as pl
from jax.experimental.pallas import tpu as pltpu
from jax.experimental.pallas import tpu_sc as plsc

N = 10000
E = 160000
D = 128
B = 128
NEG_SLOPE = 0.01

NPAD = 10240
EPAD = 163840
NC = 2
NS = 16
NW = NC * NS
EW = EPAD // NW
C1 = 64
C3 = 64
C4 = 64
QCOLS = 256


def _hdot(a, b):
  return jax.lax.dot_general(a, b, (((a.ndim - 1,), (0,)), ((), ())),
                             precision=jax.lax.Precision.HIGHEST,
                             preferred_element_type=jnp.float32)

def _t0_body(wq_ref, wk_ref, qst_ref, qr_ref, m_ref, qtab_ref, u_ref, v_ref):
  wq = wq_ref[...]
  wk = wk_ref[...]
  m = jax.lax.dot_general(wq, wk, (((0,), (0,)), ((), ())),
                          precision=jax.lax.Precision.HIGHEST,
                          preferred_element_type=jnp.float32)
  m_ref[...] = m

  qst = qst_ref[...]
  qr = qr_ref[...]

  def blk(a, b):
    return m[a * D:(a + 1) * D, b * D:(b + 1) * D]

  h2 = (_hdot(qst, blk(1, 2).T) + _hdot(qr, blk(1, 3).T)
        + _hdot(qst, blk(2, 1)) + _hdot(qr, blk(3, 1)))
  c = (jnp.sum(qst * (_hdot(qst, blk(2, 2).T) + _hdot(qr, blk(2, 3).T)), axis=1)
       + jnp.sum(qr * (_hdot(qst, blk(3, 2).T) + _hdot(qr, blk(3, 3).T)), axis=1))
  u = _hdot(qst, blk(0, 2).T) + _hdot(qr, blk(0, 3).T)
  v = _hdot(qst, blk(2, 0)) + _hdot(qr, blk(3, 0))
  qtab_ref[...] = jnp.concatenate(
      [h2, c[:, None], jnp.zeros((B, QCOLS - D - 1), jnp.float32)], axis=1)
  u_ref[...] = u
  v_ref[...] = v


def _tables_small(Wq, Wk, qst, qr):
  return pl.pallas_call(
      _t0_body,
      out_shape=(
          jax.ShapeDtypeStruct((4 * D, 4 * D), jnp.float32),
          jax.ShapeDtypeStruct((B, QCOLS), jnp.float32),
          jax.ShapeDtypeStruct((B, D), jnp.float32),
          jax.ShapeDtypeStruct((B, D), jnp.float32),
      ),
  )(Wq, Wk, qst, qr)


def _t1_body(nb_ref, m_ref, u_ref, v_ref, tsrc_ref, tdst_ref):
  nb = nb_ref[...]
  m = m_ref[...]

  def blk(a, b):
    return m[a * D:(a + 1) * D, b * D:(b + 1) * D]

  gp = _hdot(nb, blk(0, 1))
  p = _hdot(nb, blk(0, 0))
  x = _hdot(nb, u_ref[...].T)
  g = _hdot(nb, blk(1, 0).T)
  y = _hdot(nb, v_ref[...].T)
  tsrc_ref[...] = jnp.concatenate([gp, p, x], axis=1)
  tdst_ref[...] = jnp.concatenate([g, nb, y], axis=1)


def _tables_node(node_pad, M, U, V):
  grid = NPAD // 512
  return pl.pallas_call(
      _t1_body,
      grid=(grid,),
      in_specs=[
          pl.BlockSpec((512, D), lambda i: (i, 0)),
          pl.BlockSpec((4 * D, 4 * D), lambda i: (0, 0)),
          pl.BlockSpec((B, D), lambda i: (0, 0)),
          pl.BlockSpec((B, D), lambda i: (0, 0)),
      ],
      out_specs=(
          pl.BlockSpec((512, 3 * D), lambda i: (i, 0)),
          pl.BlockSpec((512, 3 * D), lambda i: (i, 0)),
      ),
      out_shape=(
          jax.ShapeDtypeStruct((NPAD, 3 * D), jnp.float32),
          jax.ShapeDtypeStruct((NPAD, 3 * D), jnp.float32),
      ),
  )(node_pad, M, U, V)


def _t2_body(rel_ref, m11_ref, z_ref):
  relb = rel_ref[...]
  m11 = m11_ref[...]
  t = jax.lax.dot_general(relb, m11, (((2,), (1,)), ((), ())),
                          precision=jax.lax.Precision.HIGHEST,
                          preferred_element_type=jnp.float32)
  z_ref[...] = jnp.sum(relb * t, axis=2)


def _quad_form(rel3, M11):
  grid = 25
  return pl.pallas_call(
      _t2_body,
      grid=(grid,),
      in_specs=[
          pl.BlockSpec((40, 160, D), lambda i: (i, 0, 0)),
          pl.BlockSpec((D, D), lambda i: (0, 0)),
      ],
      out_specs=pl.BlockSpec((40, 160), lambda i: (i, 0)),
      out_shape=jax.ShapeDtypeStruct((1000, 160), jnp.float32),
  )(rel3, M11)


def _t5_body(ag_ref, sc_ref, ms_ref, nb_ref, w_ref, b_ref, out_ref, score_ref):
  agg = ag_ref[0] + ag_ref[1]
  mask = ms_ref[...]
  upd = agg + mask * nb_ref[...]
  uu = jax.lax.bitcast_convert_type(upd, jnp.int32)
  ur = (uu + 0x7FFF + ((uu >> 16) & 1)) & jnp.int32(-65536)
  updr = jax.lax.bitcast_convert_type(ur, jnp.float32)
  out = _hdot(updr, w_ref[...].T) + b_ref[...]
  out_ref[...] = jnp.where(out >= 0.0, out, NEG_SLOPE * out)
  score_ref[...] = sc_ref[0] + sc_ref[1]


def _finalize(agg2, score2r, ms2d, node_pad, W_lin, b_lin2):
  grid = NPAD // 512
  return pl.pallas_call(
      _t5_body,
      grid=(grid,),
      in_specs=[
          pl.BlockSpec((2, 512, D), lambda i: (0, i, 0)),
          pl.BlockSpec((2, 1, 1, 512), lambda i: (0, i, 0, 0)),
          pl.BlockSpec((512, 1), lambda i: (i, 0)),
          pl.BlockSpec((512, D), lambda i: (i, 0)),
          pl.BlockSpec((D, D), lambda i: (0, 0)),
          pl.BlockSpec((1, D), lambda i: (0, 0)),
      ],
      out_specs=(
          pl.BlockSpec((512, D), lambda i: (i, 0)),
          pl.BlockSpec((1, 1, 512), lambda i: (i, 0, 0)),
      ),
      out_shape=(
          jax.ShapeDtypeStruct((NPAD, D), jnp.float32),
          jax.ShapeDtypeStruct((NPAD // 512, 1, 512), jnp.float32),
      ),
  )(agg2, score2r, ms2d, node_pad, W_lin, b_lin2)


_MESH = plsc.VectorSubcoreMesh(core_axis_name="c", subcore_axis_name="s")


def _wid():
  return lax.axis_index("s") * NC + lax.axis_index("c")


def _iota16():
  return lax.iota(jnp.int32, 16)


def _splat(x):
  return jnp.broadcast_to(x, (16,))


def _p1_body(tsrc, tdst, qtab, relh, esh, edh, qih, zh,
             logits_out, lmax_out,
             bufS, bufD, bufQ, relbuf, isrcv, idstv, iqv, eidv,
             zbuf, lgbuf, kscr, vscr, lmax):
  wid = _wid()
  ebase = wid * EW

  neg = jnp.full((16,), -1e30, jnp.float32)

  def init_body(k, _):
    lmax[pl.ds(k * 16, 16)] = neg
    return 0

  lax.fori_loop(0, NPAD // 16, init_body, 0)

  pltpu.sync_copy(qtab, bufQ)

  iota = _iota16()
  emax = _splat(E - 1)

  def round_body(r, _):
    base = ebase + r * C1
    pltpu.sync_copy(esh.at[pl.ds(base, C1)], isrcv)
    pltpu.sync_copy(edh.at[pl.ds(base, C1)], idstv)
    pltpu.sync_copy(qih.at[pl.ds(base, C1)], iqv)
    pltpu.sync_copy(zh.at[pl.ds(base, C1)], zbuf)
    for g in range(C1 // 16):
      eidv[pl.ds(g * 16, 16)] = jnp.minimum(_splat(base + g * 16) + iota, emax)
    pltpu.sync_copy(tsrc.at[isrcv], bufS)
    pltpu.sync_copy(tdst.at[idstv], bufD)
    pltpu.sync_copy(relh.at[eidv], relbuf)

    for g in range(C1 // 16):
      lane = iota + (g * 16)
      s16 = isrcv[pl.ds(g * 16, 16)]
      iq16 = iqv[pl.ds(g * 16, 16)]
      z16 = zbuf[pl.ds(g * 16, 16)]
      c16 = plsc.load_gather(bufQ, [iq16, _splat(D)])
      x16 = plsc.load_gather(bufS, [lane, _splat(2 * D) + iq16])
      y16 = plsc.load_gather(bufD, [lane, _splat(2 * D) + iq16])
      acc0 = z16 + c16 + x16 + y16

      def dot_body(j, acc):
        for jj in range(4):
          jb = _splat(j * 4 + jj)
          r16 = plsc.load_gather(relbuf, [lane, jb])
          gp16 = plsc.load_gather(bufS, [lane, jb])
          p16 = plsc.load_gather(bufS, [lane, _splat(D) + jb])
          g16 = plsc.load_gather(bufD, [lane, jb])
          nr16 = plsc.load_gather(bufD, [lane, _splat(D) + jb])
          h16 = plsc.load_gather(bufQ, [iq16, jb])
          acc = acc + r16 * (gp16 + g16 + h16) + p16 * nr16
        return acc

      acc = lax.fori_loop(0, D // 4, dot_body, acc0)
      lgbuf[pl.ds(g * 16, 16)] = acc

      key, val = plsc.sort_key_val(s16, acc)
      for sh in (1, 2, 4, 8):
        kscr[...] = key
        vscr[...] = val
        back = jnp.maximum(iota - sh, 0)
        kb = plsc.load_gather(kscr, [back])
        vb = plsc.load_gather(vscr, [back])
        same = jnp.logical_and(kb == key, iota >= sh)
        val = jnp.where(same, jnp.maximum(val, vb), val)
      kscr[...] = key
      nxt = jnp.minimum(iota + 1, 15)
      kn = plsc.load_gather(kscr, [nxt])
      is_last = jnp.logical_or(kn != key, iota == 15)
      cur = plsc.load_gather(lmax, [key])
      plsc.store_scatter(lmax, [key], jnp.maximum(cur, val), mask=is_last)

    pltpu.sync_copy(lgbuf, logits_out.at[pl.ds(base, C1)])
    return 0

  lax.fori_loop(0, EW // C1, round_body, 0)
  pltpu.sync_copy(lmax, lmax_out.at[wid])


def _run_p1(tsrc, tdst, qtab, relh, esh, edh, qih, zh):
  return pl.kernel(
      _p1_body,
      out_type=(
          jax.ShapeDtypeStruct((EPAD,), jnp.float32),
          jax.ShapeDtypeStruct((NW, NPAD), jnp.float32),
      ),
      mesh=_MESH,
      compiler_params=pltpu.CompilerParams(use_tc_tiling_on_sc=False, needs_layout_passes=False),
      scratch_types=[
          pltpu.VMEM((C1, 3 * D), jnp.float32),
          pltpu.VMEM((C1, 3 * D), jnp.float32),
          pltpu.VMEM((B, QCOLS), jnp.float32),
          pltpu.VMEM((C1, D), jnp.float32),
          pltpu.VMEM((C1,), jnp.int32),
          pltpu.VMEM((C1,), jnp.int32),
          pltpu.VMEM((C1,), jnp.int32),
          pltpu.VMEM((C1,), jnp.int32),
          pltpu.VMEM((C1,), jnp.float32),
          pltpu.VMEM((C1,), jnp.float32),
          pltpu.VMEM((16,), jnp.int32),
          pltpu.VMEM((16,), jnp.float32),
          pltpu.VMEM((NPAD,), jnp.float32),
      ],
  )(tsrc, tdst, qtab, relh, esh, edh, qih, zh)


def _p3_body(lmaxh, logitsh, esh, exh, denomh,
             gmax, mbuf, lgbuf, exbuf, isrcv, zv, denom_sp):
  cid = lax.axis_index("c")
  sid = lax.axis_index("s")
  wid = sid * NC + cid
  ebase = wid * EW

  zero = jnp.zeros((16,), jnp.float32)

  def z_body(k, _):
    zv[pl.ds(k * 16, 16)] = zero
    return 0

  lax.fori_loop(0, (NPAD // NS) // 16, z_body, 0)
  pltpu.sync_copy(zv, denom_sp.at[pl.ds(sid * (NPAD // NS), NPAD // NS)])

  def cmb_outer(kk, _):
    pltpu.sync_copy(lmaxh.at[:, pl.ds(kk * 2048, 2048)], mbuf)

    def cmb_inner(j, _):
      m = mbuf[0, pl.ds(j * 16, 16)]
      for w in range(1, NW):
        m = jnp.maximum(m, mbuf[w, pl.ds(j * 16, 16)])
      gmax[pl.ds(kk * 2048 + j * 16, 16)] = m
      return 0

    lax.fori_loop(0, 2048 // 16, cmb_inner, 0)
    return 0

  lax.fori_loop(0, NPAD // 2048, cmb_outer, 0)
  plsc.subcore_barrier()

  def round_body(r, _):
    base = ebase + r * C3
    pltpu.sync_copy(logitsh.at[pl.ds(base, C3)], lgbuf)
    pltpu.sync_copy(esh.at[pl.ds(base, C3)], isrcv)
    for g in range(C3 // 16):
      lg16 = lgbuf[pl.ds(g * 16, 16)]
      s16 = isrcv[pl.ds(g * 16, 16)]
      gm16 = plsc.load_gather(gmax, [s16])
      exbuf[pl.ds(g * 16, 16)] = jnp.exp(lg16 - gm16)
    pltpu.sync_copy(exbuf, exh.at[pl.ds(base, C3)])
    pltpu.sync_copy(exbuf, denom_sp.at[isrcv], add=True)
    return 0

  lax.fori_loop(0, EW // C3, round_body, 0)
  plsc.subcore_barrier()
  sl = pl.ds(sid * (NPAD // NS), NPAD // NS)
  pltpu.sync_copy(denom_sp.at[sl], denomh.at[cid, sl])


def _run_p3(lmaxh, logitsh, esh):
  return pl.kernel(
      _p3_body,
      out_type=(
          jax.ShapeDtypeStruct((EPAD,), jnp.float32),
          jax.ShapeDtypeStruct((NC, NPAD), jnp.float32),
      ),
      mesh=_MESH,
      compiler_params=pltpu.CompilerParams(use_tc_tiling_on_sc=False, needs_layout_passes=False),
      scratch_types=[
          pltpu.VMEM((NPAD,), jnp.float32),
          pltpu.VMEM((NW, 2048), jnp.float32),
          pltpu.VMEM((C3,), jnp.float32),
          pltpu.VMEM((C3,), jnp.float32),
          pltpu.VMEM((C3,), jnp.int32),
          pltpu.VMEM((NPAD // NS,), jnp.float32),
          pltpu.VMEM_SHARED((NPAD,), jnp.float32),
      ],
  )(lmaxh, logitsh, esh)


def _p4_body(exh, esh, edh, denomh, vnsh, nodeh,
             scoreh, aggh, msh,
             denv, vnsv, dbuf, exbuf, smbuf, sbuf, isrcv, idstv,
             ndbuf, zrows, msv, score_sp, agg_sp, semN):
  cid = lax.axis_index("c")
  sid = lax.axis_index("s")
  wid = sid * NC + cid
  ebase = wid * EW
  rows = NPAD // NS

  zero = jnp.zeros((16,), jnp.float32)

  def zr_body(k, _):
    for jj in range(D // 16):
      zrows[k, pl.ds(jj * 16, 16)] = zero
    return 0

  lax.fori_loop(0, 40, zr_body, 0)

  def zv_body(k, _):
    msv[pl.ds(k * 16, 16)] = zero
    return 0

  lax.fori_loop(0, rows // 16, zv_body, 0)
  pltpu.sync_copy(msv, score_sp.at[pl.ds(sid * rows, rows)])
  for t in range(16):
    pltpu.sync_copy(zrows, agg_sp.at[pl.ds(sid * rows + t * 40, 40), :])

  def den_outer(kk, _):
    pltpu.sync_copy(denomh.at[:, pl.ds(kk * 2048, 2048)], dbuf)

    def den_inner(j, _):
      denv[pl.ds(kk * 2048 + j * 16, 16)] = (
          dbuf[0, pl.ds(j * 16, 16)] + dbuf[1, pl.ds(j * 16, 16)])
      return 0

    lax.fori_loop(0, 2048 // 16, den_inner, 0)
    return 0

  lax.fori_loop(0, NPAD // 2048, den_outer, 0)
  pltpu.sync_copy(vnsh, vnsv)
  plsc.subcore_barrier()

  iota = _iota16()

  def round_body(r, _):
    base = ebase + r * C4
    pltpu.sync_copy(exh.at[pl.ds(base, C4)], exbuf)
    pltpu.sync_copy(esh.at[pl.ds(base, C4)], isrcv)
    pltpu.sync_copy(edh.at[pl.ds(base, C4)], idstv)
    dN = pltpu.async_copy(nodeh.at[idstv], ndbuf, semN)
    sms = []
    for g in range(C4 // 16):
      e16 = exbuf[pl.ds(g * 16, 16)]
      s16 = isrcv[pl.ds(g * 16, 16)]
      den16 = plsc.load_gather(denv, [s16])
      sm16 = e16 / (den16 + 1e-16)
      vn16 = plsc.load_gather(vnsv, [s16])
      sbuf[pl.ds(g * 16, 16)] = sm16 * vn16
      smbuf[pl.ds(g * 16, 16)] = sm16
      sms.append(sm16)
    dN.wait()

    def sc_body(j, _):
      jb = _splat(j)
      for g in range(C4 // 16):
        lane = iota + (g * 16)
        val = plsc.load_gather(ndbuf, [lane, jb]) * sms[g]
        plsc.store_scatter(ndbuf, [lane, jb], val)
      return 0

    lax.fori_loop(0, D, sc_body, 0)
    pltpu.sync_copy(ndbuf, agg_sp.at[isrcv], add=True)
    pltpu.sync_copy(sbuf, score_sp.at[idstv], add=True)
    return 0

  lax.fori_loop(0, EW // C4, round_body, 0)

  def ms_body(k, _):
    d16 = denv[pl.ds(sid * rows + k * 16, 16)]
    msv[pl.ds(k * 16, 16)] = jnp.where(d16 > 0.0, 0.0, 1.0)
    return 0

  lax.fori_loop(0, rows // 16, ms_body, 0)

  plsc.subcore_barrier()
  sl = pl.ds(sid * rows, rows)
  pltpu.sync_copy(score_sp.at[sl], scoreh.at[cid, sl])
  pltpu.sync_copy(agg_sp.at[sl, :], aggh.at[cid, sl, :])

  @pl.when(cid == 0)
  def _():
    pltpu.sync_copy(msv, msh.at[sl])


def _run_p4(exh, esh, edh, denomh, vnsh, nodeh):
  return pl.kernel(
      _p4_body,
      out_type=(
          jax.ShapeDtypeStruct((NC, NPAD), jnp.float32),
          jax.ShapeDtypeStruct((NC, NPAD, D), jnp.float32),
          jax.ShapeDtypeStruct((NPAD,), jnp.float32),
      ),
      mesh=_MESH,
      compiler_params=pltpu.CompilerParams(use_tc_tiling_on_sc=False, needs_layout_passes=False),
      scratch_types=[
          pltpu.VMEM((NPAD,), jnp.float32),
          pltpu.VMEM((NPAD,), jnp.float32),
          pltpu.VMEM((NC, 2048), jnp.float32),
          pltpu.VMEM((C4,), jnp.float32),
          pltpu.VMEM((C4,), jnp.float32),
          pltpu.VMEM((C4,), jnp.float32),
          pltpu.VMEM((C4,), jnp.int32),
          pltpu.VMEM((C4,), jnp.int32),
          pltpu.VMEM((C4, D), jnp.float32),
          pltpu.VMEM((40, D), jnp.float32),
          pltpu.VMEM((NPAD // NS,), jnp.float32),
          pltpu.VMEM_SHARED((NPAD,), jnp.float32),
          pltpu.VMEM_SHARED((NPAD, D), jnp.float32),
          pltpu.SemaphoreType.DMA,
      ],
  )(exh, esh, edh, denomh, vnsh, nodeh)


@jax.jit
def kernel(node_repr, rel_emb, query_src_ts_emb, query_rel_emb,
           visited_node_score, Wq, Wk, W_lin, b_lin,
           edge_src, edge_dst, query_idx):
  def _r(x):
    u = jax.lax.bitcast_convert_type(x, jnp.int32)
    r = (u + 0x7FFF + ((u >> 16) & 1)) & jnp.int32(-65536)
    return jax.lax.bitcast_convert_type(r, jnp.float32)

  node_pad = jnp.concatenate(
      [node_repr, jnp.zeros((NPAD - N, D), jnp.float32)], axis=0)
  node_pad_r = _r(node_pad)
  rel_r = _r(rel_emb)
  vns_pad = jnp.concatenate(
      [visited_node_score, jnp.zeros((NPAD - N,), jnp.float32)])
  pad_i = jnp.full((EPAD - E,), N, jnp.int32)
  es_pad = jnp.concatenate([edge_src, pad_i])
  ed_pad = jnp.concatenate([edge_dst, pad_i])
  qi_pad = jnp.concatenate([query_idx, jnp.zeros((EPAD - E,), jnp.int32)])

  M, qtab, U, V = _tables_small(_r(Wq), _r(Wk), _r(query_src_ts_emb),
                                _r(query_rel_emb))
  tsrc, tdst = _tables_node(node_pad_r, M, U, V)
  z2d = _quad_form(rel_r.reshape(1000, 160, D), M[D:2 * D, D:2 * D])
  z_pad = jnp.concatenate(
      [z2d.reshape(E), jnp.zeros((EPAD - E,), jnp.float32)])

  logits, lmaxh = _run_p1(tsrc, tdst, qtab, rel_r,
                          es_pad, ed_pad, qi_pad, z_pad)
  exh, denomh = _run_p3(lmaxh, logits, es_pad)
  scoreh, aggh, msh = _run_p4(exh, es_pad, ed_pad, denomh, vns_pad, node_pad)

  out_repr_pad, score2d = _finalize(
      aggh, scoreh.reshape(NC, NPAD // 512, 1, 512), msh.reshape(NPAD, 1),
      node_pad, _r(W_lin), b_lin.reshape(1, D))
  return score2d.reshape(NPAD)[:N], out_repr_pad[:N]


# speedup vs baseline: 1.5217x; 1.2265x over previous
import jax
import jax.numpy as jnp
from jax import lax
from jax.experimental import pallas as pl
from jax.experimental.pallas import tpu as pltpu
from jax.experimental.pallas import tpu_sc as plsc

N = 10000
E = 160000
D = 128
B = 128
NEG_SLOPE = 0.01

NPAD = 10240
EPAD = 163840
NC = 2
NS = 16
NW = NC * NS
EW = EPAD // NW
C1 = 32
C3 = 64
C4 = 64
QCOLS = 256


def _hdot(a, b):
  return jax.lax.dot_general(a, b, (((a.ndim - 1,), (0,)), ((), ())),
                             precision=jax.lax.Precision.HIGHEST,
                             preferred_element_type=jnp.float32)

def _t0_body(wq_ref, wk_ref, qst_ref, qr_ref, m_ref, qtab_ref, u_ref, v_ref):
  wq = wq_ref[...]
  wk = wk_ref[...]
  m = jax.lax.dot_general(wq, wk, (((0,), (0,)), ((), ())),
                          precision=jax.lax.Precision.HIGHEST,
                          preferred_element_type=jnp.float32)
  m_ref[...] = m

  qst = qst_ref[...]
  qr = qr_ref[...]

  def blk(a, b):
    return m[a * D:(a + 1) * D, b * D:(b + 1) * D]

  h2 = (_hdot(qst, blk(1, 2).T) + _hdot(qr, blk(1, 3).T)
        + _hdot(qst, blk(2, 1)) + _hdot(qr, blk(3, 1)))
  c = (jnp.sum(qst * (_hdot(qst, blk(2, 2).T) + _hdot(qr, blk(2, 3).T)), axis=1)
       + jnp.sum(qr * (_hdot(qst, blk(3, 2).T) + _hdot(qr, blk(3, 3).T)), axis=1))
  u = _hdot(qst, blk(0, 2).T) + _hdot(qr, blk(0, 3).T)
  v = _hdot(qst, blk(2, 0)) + _hdot(qr, blk(3, 0))
  qtab_ref[...] = jnp.concatenate(
      [h2, c[:, None], jnp.zeros((B, QCOLS - D - 1), jnp.float32)], axis=1)
  u_ref[...] = u
  v_ref[...] = v


def _tables_small(Wq, Wk, qst, qr):
  return pl.pallas_call(
      _t0_body,
      out_shape=(
          jax.ShapeDtypeStruct((4 * D, 4 * D), jnp.float32),
          jax.ShapeDtypeStruct((B, QCOLS), jnp.float32),
          jax.ShapeDtypeStruct((B, D), jnp.float32),
          jax.ShapeDtypeStruct((B, D), jnp.float32),
      ),
  )(Wq, Wk, qst, qr)


def _t1_body(nb_ref, m_ref, u_ref, v_ref, tsrc_ref, tdst_ref):
  nb = nb_ref[...]
  m = m_ref[...]

  def blk(a, b):
    return m[a * D:(a + 1) * D, b * D:(b + 1) * D]

  gp = _hdot(nb, blk(0, 1))
  p = _hdot(nb, blk(0, 0))
  x = _hdot(nb, u_ref[...].T)
  g = _hdot(nb, blk(1, 0).T)
  y = _hdot(nb, v_ref[...].T)
  tsrc_ref[...] = jnp.concatenate([gp, p, x], axis=1)
  tdst_ref[...] = jnp.concatenate([g, nb, y], axis=1)


def _tables_node(node_pad, M, U, V):
  grid = NPAD // 512
  return pl.pallas_call(
      _t1_body,
      grid=(grid,),
      in_specs=[
          pl.BlockSpec((512, D), lambda i: (i, 0)),
          pl.BlockSpec((4 * D, 4 * D), lambda i: (0, 0)),
          pl.BlockSpec((B, D), lambda i: (0, 0)),
          pl.BlockSpec((B, D), lambda i: (0, 0)),
      ],
      out_specs=(
          pl.BlockSpec((512, 3 * D), lambda i: (i, 0)),
          pl.BlockSpec((512, 3 * D), lambda i: (i, 0)),
      ),
      out_shape=(
          jax.ShapeDtypeStruct((NPAD, 3 * D), jnp.float32),
          jax.ShapeDtypeStruct((NPAD, 3 * D), jnp.float32),
      ),
  )(node_pad, M, U, V)


def _t2_body(rel_ref, m11_ref, z_ref):
  relb = rel_ref[...]
  m11 = m11_ref[...]
  t = jax.lax.dot_general(relb, m11, (((2,), (1,)), ((), ())),
                          precision=jax.lax.Precision.HIGHEST,
                          preferred_element_type=jnp.float32)
  z_ref[...] = jnp.sum(relb * t, axis=2)


def _quad_form(rel3, M11):
  grid = 25
  return pl.pallas_call(
      _t2_body,
      grid=(grid,),
      in_specs=[
          pl.BlockSpec((40, 160, D), lambda i: (i, 0, 0)),
          pl.BlockSpec((D, D), lambda i: (0, 0)),
      ],
      out_specs=pl.BlockSpec((40, 160), lambda i: (i, 0)),
      out_shape=jax.ShapeDtypeStruct((1000, 160), jnp.float32),
  )(rel3, M11)


def _t5_body(ag_ref, sc_ref, ms_ref, nb_ref, w_ref, b_ref, out_ref, score_ref):
  agg = ag_ref[0] + ag_ref[1]
  mask = ms_ref[...]
  upd = agg + mask * nb_ref[...]
  uu = jax.lax.bitcast_convert_type(upd, jnp.int32)
  ur = (uu + 0x7FFF + ((uu >> 16) & 1)) & jnp.int32(-65536)
  updr = jax.lax.bitcast_convert_type(ur, jnp.float32)
  out = _hdot(updr, w_ref[...].T) + b_ref[...]
  out_ref[...] = jnp.where(out >= 0.0, out, NEG_SLOPE * out)
  score_ref[...] = sc_ref[0] + sc_ref[1]


def _finalize(agg2, score2r, ms2d, node_pad, W_lin, b_lin2):
  grid = NPAD // 512
  return pl.pallas_call(
      _t5_body,
      grid=(grid,),
      in_specs=[
          pl.BlockSpec((2, 512, D), lambda i: (0, i, 0)),
          pl.BlockSpec((2, 1, 1, 512), lambda i: (0, i, 0, 0)),
          pl.BlockSpec((512, 1), lambda i: (i, 0)),
          pl.BlockSpec((512, D), lambda i: (i, 0)),
          pl.BlockSpec((D, D), lambda i: (0, 0)),
          pl.BlockSpec((1, D), lambda i: (0, 0)),
      ],
      out_specs=(
          pl.BlockSpec((512, D), lambda i: (i, 0)),
          pl.BlockSpec((1, 1, 512), lambda i: (i, 0, 0)),
      ),
      out_shape=(
          jax.ShapeDtypeStruct((NPAD, D), jnp.float32),
          jax.ShapeDtypeStruct((NPAD // 512, 1, 512), jnp.float32),
      ),
  )(agg2, score2r, ms2d, node_pad, W_lin, b_lin2)


_MESH = plsc.VectorSubcoreMesh(core_axis_name="c", subcore_axis_name="s")


def _wid():
  return lax.axis_index("s") * NC + lax.axis_index("c")


def _iota16():
  return lax.iota(jnp.int32, 16)


def _splat(x):
  return jnp.broadcast_to(x, (16,))


def _p1_body(tsrc, tdst, qtab, relh, esh, edh, qih, zh,
             logits_out, lmax_out,
             bufS0, bufS1, bufD0, bufD1, relb0, relb1, eid0, eid1,
             esv, edv, qiv, zv, bufQ, lgbuf, kscr, vscr, lmax,
             smS0, smS1, smD0, smD1, smR0, smR1):
  wid = _wid()
  ebase = wid * EW
  bufS_ = (bufS0, bufS1)
  bufD_ = (bufD0, bufD1)
  relb_ = (relb0, relb1)
  eid_ = (eid0, eid1)
  smS_ = (smS0, smS1)
  smD_ = (smD0, smD1)
  smR_ = (smR0, smR1)
  ROUNDS = EW // C1

  neg = jnp.full((16,), -1e30, jnp.float32)

  def init_body(k, _):
    lmax[pl.ds(k * 16, 16)] = neg
    return 0

  lax.fori_loop(0, NPAD // 16, init_body, 0)

  pltpu.sync_copy(qtab, bufQ)
  pltpu.sync_copy(esh.at[pl.ds(ebase, EW)], esv)
  pltpu.sync_copy(edh.at[pl.ds(ebase, EW)], edv)
  pltpu.sync_copy(qih.at[pl.ds(ebase, EW)], qiv)
  pltpu.sync_copy(zh.at[pl.ds(ebase, EW)], zv)

  iota = _iota16()
  emax = _splat(E - 1)

  def issue(r, b):
    lb = r * C1
    for g in range(C1 // 16):
      eid_[b][pl.ds(g * 16, 16)] = jnp.minimum(
          _splat(ebase + lb + g * 16) + iota, emax)
    pltpu.async_copy(tsrc.at[esv.at[pl.ds(lb, C1)]], bufS_[b], smS_[b])
    pltpu.async_copy(tdst.at[edv.at[pl.ds(lb, C1)]], bufD_[b], smD_[b])
    pltpu.async_copy(relh.at[eid_[b]], relb_[b], smR_[b])

  def drain(b):
    pltpu.make_async_copy(tsrc.at[pl.ds(0, C1)], bufS_[b], smS_[b]).wait()
    pltpu.make_async_copy(tdst.at[pl.ds(0, C1)], bufD_[b], smD_[b]).wait()
    pltpu.make_async_copy(relh.at[pl.ds(0, C1)], relb_[b], smR_[b]).wait()

  def compute(r, b):
    lb = r * C1
    bufS = bufS_[b]
    bufD = bufD_[b]
    relbuf = relb_[b]
    for g in range(C1 // 16):
      lane = iota + (g * 16)
      s16 = esv[pl.ds(lb + g * 16, 16)]
      iq16 = qiv[pl.ds(lb + g * 16, 16)]
      z16 = zv[pl.ds(lb + g * 16, 16)]
      c16 = plsc.load_gather(bufQ, [iq16, _splat(D)])
      x16 = plsc.load_gather(bufS, [lane, _splat(2 * D) + iq16])
      y16 = plsc.load_gather(bufD, [lane, _splat(2 * D) + iq16])
      acc0 = z16 + c16 + x16 + y16

      def dot_body(j, acc):
        for jj in range(4):
          jb = _splat(j * 4 + jj)
          r16 = plsc.load_gather(relbuf, [lane, jb])
          gp16 = plsc.load_gather(bufS, [lane, jb])
          p16 = plsc.load_gather(bufS, [lane, _splat(D) + jb])
          g16 = plsc.load_gather(bufD, [lane, jb])
          nr16 = plsc.load_gather(bufD, [lane, _splat(D) + jb])
          h16 = plsc.load_gather(bufQ, [iq16, jb])
          acc = acc + r16 * (gp16 + g16 + h16) + p16 * nr16
        return acc

      acc = lax.fori_loop(0, D // 4, dot_body, acc0)
      lgbuf[pl.ds(g * 16, 16)] = acc

      key, val = plsc.sort_key_val(s16, acc)
      for sh in (1, 2, 4, 8):
        kscr[...] = key
        vscr[...] = val
        back = jnp.maximum(iota - sh, 0)
        kb = plsc.load_gather(kscr, [back])
        vb = plsc.load_gather(vscr, [back])
        same = jnp.logical_and(kb == key, iota >= sh)
        val = jnp.where(same, jnp.maximum(val, vb), val)
      kscr[...] = key
      nxt = jnp.minimum(iota + 1, 15)
      kn = plsc.load_gather(kscr, [nxt])
      is_last = jnp.logical_or(kn != key, iota == 15)
      cur = plsc.load_gather(lmax, [key])
      plsc.store_scatter(lmax, [key], jnp.maximum(cur, val), mask=is_last)

    pltpu.sync_copy(lgbuf, logits_out.at[pl.ds(ebase + lb, C1)])

  issue(jnp.int32(0), 0)
  last = jnp.int32(ROUNDS - 1)

  def pair_body(h, _):
    r0 = h * 2
    drain(0)
    issue(r0 + 1, 1)
    compute(r0, 0)
    drain(1)
    issue(jnp.minimum(r0 + 2, last), 0)
    compute(r0 + 1, 1)
    return 0

  lax.fori_loop(0, ROUNDS // 2, pair_body, 0)
  drain(0)
  pltpu.sync_copy(lmax, lmax_out.at[wid])


def _run_p1(tsrc, tdst, qtab, relh, esh, edh, qih, zh):
  return pl.kernel(
      _p1_body,
      out_type=(
          jax.ShapeDtypeStruct((EPAD,), jnp.float32),
          jax.ShapeDtypeStruct((NW, NPAD), jnp.float32),
      ),
      mesh=_MESH,
      compiler_params=pltpu.CompilerParams(use_tc_tiling_on_sc=False, needs_layout_passes=False),
      scratch_types=[
          pltpu.VMEM((C1, 3 * D), jnp.float32),
          pltpu.VMEM((C1, 3 * D), jnp.float32),
          pltpu.VMEM((C1, 3 * D), jnp.float32),
          pltpu.VMEM((C1, 3 * D), jnp.float32),
          pltpu.VMEM((C1, D), jnp.float32),
          pltpu.VMEM((C1, D), jnp.float32),
          pltpu.VMEM((C1,), jnp.int32),
          pltpu.VMEM((C1,), jnp.int32),
          pltpu.VMEM((EW,), jnp.int32),
          pltpu.VMEM((EW,), jnp.int32),
          pltpu.VMEM((EW,), jnp.int32),
          pltpu.VMEM((EW,), jnp.float32),
          pltpu.VMEM((B, QCOLS), jnp.float32),
          pltpu.VMEM((C1,), jnp.float32),
          pltpu.VMEM((16,), jnp.int32),
          pltpu.VMEM((16,), jnp.float32),
          pltpu.VMEM((NPAD,), jnp.float32),
          pltpu.SemaphoreType.DMA,
          pltpu.SemaphoreType.DMA,
          pltpu.SemaphoreType.DMA,
          pltpu.SemaphoreType.DMA,
          pltpu.SemaphoreType.DMA,
          pltpu.SemaphoreType.DMA,
      ],
  )(tsrc, tdst, qtab, relh, esh, edh, qih, zh)


def _p3_body(lmaxh, logitsh, esh, exh, denomh,
             gmax, mbuf, lgbuf, exbuf, isrcv, zv, denom_sp):
  cid = lax.axis_index("c")
  sid = lax.axis_index("s")
  wid = sid * NC + cid
  ebase = wid * EW

  zero = jnp.zeros((16,), jnp.float32)

  def z_body(k, _):
    zv[pl.ds(k * 16, 16)] = zero
    return 0

  lax.fori_loop(0, (NPAD // NS) // 16, z_body, 0)
  pltpu.sync_copy(zv, denom_sp.at[pl.ds(sid * (NPAD // NS), NPAD // NS)])

  def cmb_outer(kk, _):
    pltpu.sync_copy(lmaxh.at[:, pl.ds(kk * 2048, 2048)], mbuf)

    def cmb_inner(j, _):
      m = mbuf[0, pl.ds(j * 16, 16)]
      for w in range(1, NW):
        m = jnp.maximum(m, mbuf[w, pl.ds(j * 16, 16)])
      gmax[pl.ds(kk * 2048 + j * 16, 16)] = m
      return 0

    lax.fori_loop(0, 2048 // 16, cmb_inner, 0)
    return 0

  lax.fori_loop(0, NPAD // 2048, cmb_outer, 0)
  plsc.subcore_barrier()

  def round_body(r, _):
    base = ebase + r * C3
    pltpu.sync_copy(logitsh.at[pl.ds(base, C3)], lgbuf)
    pltpu.sync_copy(esh.at[pl.ds(base, C3)], isrcv)
    for g in range(C3 // 16):
      lg16 = lgbuf[pl.ds(g * 16, 16)]
      s16 = isrcv[pl.ds(g * 16, 16)]
      gm16 = plsc.load_gather(gmax, [s16])
      exbuf[pl.ds(g * 16, 16)] = jnp.exp(lg16 - gm16)
    pltpu.sync_copy(exbuf, exh.at[pl.ds(base, C3)])
    pltpu.sync_copy(exbuf, denom_sp.at[isrcv], add=True)
    return 0

  lax.fori_loop(0, EW // C3, round_body, 0)
  plsc.subcore_barrier()
  sl = pl.ds(sid * (NPAD // NS), NPAD // NS)
  pltpu.sync_copy(denom_sp.at[sl], denomh.at[cid, sl])


def _run_p3(lmaxh, logitsh, esh):
  return pl.kernel(
      _p3_body,
      out_type=(
          jax.ShapeDtypeStruct((EPAD,), jnp.float32),
          jax.ShapeDtypeStruct((NC, NPAD), jnp.float32),
      ),
      mesh=_MESH,
      compiler_params=pltpu.CompilerParams(use_tc_tiling_on_sc=False, needs_layout_passes=False),
      scratch_types=[
          pltpu.VMEM((NPAD,), jnp.float32),
          pltpu.VMEM((NW, 2048), jnp.float32),
          pltpu.VMEM((C3,), jnp.float32),
          pltpu.VMEM((C3,), jnp.float32),
          pltpu.VMEM((C3,), jnp.int32),
          pltpu.VMEM((NPAD // NS,), jnp.float32),
          pltpu.VMEM_SHARED((NPAD,), jnp.float32),
      ],
  )(lmaxh, logitsh, esh)


def _p4_body(exh, esh, edh, denomh, vnsh, nodeh,
             scoreh, aggh, msh,
             denv, vnsv, dbuf, exbuf, smbuf, sbuf, isrcv, idstv,
             ndbuf, zrows, msv, score_sp, agg_sp, semN):
  cid = lax.axis_index("c")
  sid = lax.axis_index("s")
  wid = sid * NC + cid
  ebase = wid * EW
  rows = NPAD // NS

  zero = jnp.zeros((16,), jnp.float32)

  def zr_body(k, _):
    for jj in range(D // 16):
      zrows[k, pl.ds(jj * 16, 16)] = zero
    return 0

  lax.fori_loop(0, 40, zr_body, 0)

  def zv_body(k, _):
    msv[pl.ds(k * 16, 16)] = zero
    return 0

  lax.fori_loop(0, rows // 16, zv_body, 0)
  pltpu.sync_copy(msv, score_sp.at[pl.ds(sid * rows, rows)])
  for t in range(16):
    pltpu.sync_copy(zrows, agg_sp.at[pl.ds(sid * rows + t * 40, 40), :])

  def den_outer(kk, _):
    pltpu.sync_copy(denomh.at[:, pl.ds(kk * 2048, 2048)], dbuf)

    def den_inner(j, _):
      denv[pl.ds(kk * 2048 + j * 16, 16)] = (
          dbuf[0, pl.ds(j * 16, 16)] + dbuf[1, pl.ds(j * 16, 16)])
      return 0

    lax.fori_loop(0, 2048 // 16, den_inner, 0)
    return 0

  lax.fori_loop(0, NPAD // 2048, den_outer, 0)
  pltpu.sync_copy(vnsh, vnsv)
  plsc.subcore_barrier()

  iota = _iota16()

  def round_body(r, _):
    base = ebase + r * C4
    pltpu.sync_copy(exh.at[pl.ds(base, C4)], exbuf)
    pltpu.sync_copy(esh.at[pl.ds(base, C4)], isrcv)
    pltpu.sync_copy(edh.at[pl.ds(base, C4)], idstv)
    dN = pltpu.async_copy(nodeh.at[idstv], ndbuf, semN)
    sms = []
    for g in range(C4 // 16):
      e16 = exbuf[pl.ds(g * 16, 16)]
      s16 = isrcv[pl.ds(g * 16, 16)]
      den16 = plsc.load_gather(denv, [s16])
      sm16 = e16 / (den16 + 1e-16)
      vn16 = plsc.load_gather(vnsv, [s16])
      sbuf[pl.ds(g * 16, 16)] = sm16 * vn16
      smbuf[pl.ds(g * 16, 16)] = sm16
      sms.append(sm16)
    dN.wait()

    def sc_body(j, _):
      jb = _splat(j)
      for g in range(C4 // 16):
        lane = iota + (g * 16)
        val = plsc.load_gather(ndbuf, [lane, jb]) * sms[g]
        plsc.store_scatter(ndbuf, [lane, jb], val)
      return 0

    lax.fori_loop(0, D, sc_body, 0)
    pltpu.sync_copy(ndbuf, agg_sp.at[isrcv], add=True)
    pltpu.sync_copy(sbuf, score_sp.at[idstv], add=True)
    return 0

  lax.fori_loop(0, EW // C4, round_body, 0)

  def ms_body(k, _):
    d16 = denv[pl.ds(sid * rows + k * 16, 16)]
    msv[pl.ds(k * 16, 16)] = jnp.where(d16 > 0.0, 0.0, 1.0)
    return 0

  lax.fori_loop(0, rows // 16, ms_body, 0)

  plsc.subcore_barrier()
  sl = pl.ds(sid * rows, rows)
  pltpu.sync_copy(score_sp.at[sl], scoreh.at[cid, sl])
  pltpu.sync_copy(agg_sp.at[sl, :], aggh.at[cid, sl, :])

  @pl.when(cid == 0)
  def _():
    pltpu.sync_copy(msv, msh.at[sl])


def _run_p4(exh, esh, edh, denomh, vnsh, nodeh):
  return pl.kernel(
      _p4_body,
      out_type=(
          jax.ShapeDtypeStruct((NC, NPAD), jnp.float32),
          jax.ShapeDtypeStruct((NC, NPAD, D), jnp.float32),
          jax.ShapeDtypeStruct((NPAD,), jnp.float32),
      ),
      mesh=_MESH,
      compiler_params=pltpu.CompilerParams(use_tc_tiling_on_sc=False, needs_layout_passes=False),
      scratch_types=[
          pltpu.VMEM((NPAD,), jnp.float32),
          pltpu.VMEM((NPAD,), jnp.float32),
          pltpu.VMEM((NC, 2048), jnp.float32),
          pltpu.VMEM((C4,), jnp.float32),
          pltpu.VMEM((C4,), jnp.float32),
          pltpu.VMEM((C4,), jnp.float32),
          pltpu.VMEM((C4,), jnp.int32),
          pltpu.VMEM((C4,), jnp.int32),
          pltpu.VMEM((C4, D), jnp.float32),
          pltpu.VMEM((40, D), jnp.float32),
          pltpu.VMEM((NPAD // NS,), jnp.float32),
          pltpu.VMEM_SHARED((NPAD,), jnp.float32),
          pltpu.VMEM_SHARED((NPAD, D), jnp.float32),
          pltpu.SemaphoreType.DMA,
      ],
  )(exh, esh, edh, denomh, vnsh, nodeh)


@jax.jit
def kernel(node_repr, rel_emb, query_src_ts_emb, query_rel_emb,
           visited_node_score, Wq, Wk, W_lin, b_lin,
           edge_src, edge_dst, query_idx):
  def _r(x):
    u = jax.lax.bitcast_convert_type(x, jnp.int32)
    r = (u + 0x7FFF + ((u >> 16) & 1)) & jnp.int32(-65536)
    return jax.lax.bitcast_convert_type(r, jnp.float32)

  node_pad = jnp.concatenate(
      [node_repr, jnp.zeros((NPAD - N, D), jnp.float32)], axis=0)
  node_pad_r = _r(node_pad)
  rel_r = _r(rel_emb)
  vns_pad = jnp.concatenate(
      [visited_node_score, jnp.zeros((NPAD - N,), jnp.float32)])
  pad_i = jnp.full((EPAD - E,), N, jnp.int32)
  es_pad = jnp.concatenate([edge_src, pad_i])
  ed_pad = jnp.concatenate([edge_dst, pad_i])
  qi_pad = jnp.concatenate([query_idx, jnp.zeros((EPAD - E,), jnp.int32)])

  M, qtab, U, V = _tables_small(_r(Wq), _r(Wk), _r(query_src_ts_emb),
                                _r(query_rel_emb))
  tsrc, tdst = _tables_node(node_pad_r, M, U, V)
  z2d = _quad_form(rel_r.reshape(1000, 160, D), M[D:2 * D, D:2 * D])
  z_pad = jnp.concatenate(
      [z2d.reshape(E), jnp.zeros((EPAD - E,), jnp.float32)])

  logits, lmaxh = _run_p1(tsrc, tdst, qtab, rel_r,
                          es_pad, ed_pad, qi_pad, z_pad)
  exh, denomh = _run_p3(lmaxh, logits, es_pad)
  scoreh, aggh, msh = _run_p4(exh, es_pad, ed_pad, denomh, vns_pad, node_pad)

  out_repr_pad, score2d = _finalize(
      aggh, scoreh.reshape(NC, NPAD // 512, 1, 512), msh.reshape(NPAD, 1),
      node_pad, _r(W_lin), b_lin.reshape(1, D))
  return score2d.reshape(NPAD)[:N], out_repr_pad[:N]


# speedup vs baseline: 1.6430x; 1.0797x over previous
import jax
import jax.numpy as jnp
from jax import lax
from jax.experimental import pallas as pl
from jax.experimental.pallas import tpu as pltpu
from jax.experimental.pallas import tpu_sc as plsc

N = 10000
E = 160000
D = 128
B = 128
NEG_SLOPE = 0.01

NPAD = 10240
EPAD = 163840
NC = 2
NS = 16
NW = NC * NS
EW = EPAD // NW
C1 = 32
C3 = 64
C4 = 64
QCOLS = 256


def _hdot(a, b):
  return jax.lax.dot_general(a, b, (((a.ndim - 1,), (0,)), ((), ())),
                             precision=jax.lax.Precision.HIGHEST,
                             preferred_element_type=jnp.float32)

def _t0_body(wq_ref, wk_ref, qst_ref, qr_ref, m_ref, qtab_ref, u_ref, v_ref):
  wq = wq_ref[...]
  wk = wk_ref[...]
  m = jax.lax.dot_general(wq, wk, (((0,), (0,)), ((), ())),
                          precision=jax.lax.Precision.HIGHEST,
                          preferred_element_type=jnp.float32)
  m_ref[...] = m

  qst = qst_ref[...]
  qr = qr_ref[...]

  def blk(a, b):
    return m[a * D:(a + 1) * D, b * D:(b + 1) * D]

  h2 = (_hdot(qst, blk(1, 2).T) + _hdot(qr, blk(1, 3).T)
        + _hdot(qst, blk(2, 1)) + _hdot(qr, blk(3, 1)))
  c = (jnp.sum(qst * (_hdot(qst, blk(2, 2).T) + _hdot(qr, blk(2, 3).T)), axis=1)
       + jnp.sum(qr * (_hdot(qst, blk(3, 2).T) + _hdot(qr, blk(3, 3).T)), axis=1))
  u = _hdot(qst, blk(0, 2).T) + _hdot(qr, blk(0, 3).T)
  v = _hdot(qst, blk(2, 0)) + _hdot(qr, blk(3, 0))
  qtab_ref[...] = jnp.concatenate(
      [h2, c[:, None], jnp.zeros((B, QCOLS - D - 1), jnp.float32)], axis=1)
  u_ref[...] = u
  v_ref[...] = v


def _tables_small(Wq, Wk, qst, qr):
  return pl.pallas_call(
      _t0_body,
      out_shape=(
          jax.ShapeDtypeStruct((4 * D, 4 * D), jnp.float32),
          jax.ShapeDtypeStruct((B, QCOLS), jnp.float32),
          jax.ShapeDtypeStruct((B, D), jnp.float32),
          jax.ShapeDtypeStruct((B, D), jnp.float32),
      ),
  )(Wq, Wk, qst, qr)


def _t1_body(nb_ref, m_ref, u_ref, v_ref, tsrc_ref, tdst_ref):
  nb = nb_ref[...]
  m = m_ref[...]

  def blk(a, b):
    return m[a * D:(a + 1) * D, b * D:(b + 1) * D]

  gp = _hdot(nb, blk(0, 1))
  p = _hdot(nb, blk(0, 0))
  x = _hdot(nb, u_ref[...].T)
  g = _hdot(nb, blk(1, 0).T)
  y = _hdot(nb, v_ref[...].T)
  tsrc_ref[...] = jnp.concatenate([gp, p, x], axis=1)
  tdst_ref[...] = jnp.concatenate([g, nb, y], axis=1)


def _tables_node(node_pad, M, U, V):
  grid = NPAD // 512
  return pl.pallas_call(
      _t1_body,
      grid=(grid,),
      in_specs=[
          pl.BlockSpec((512, D), lambda i: (i, 0)),
          pl.BlockSpec((4 * D, 4 * D), lambda i: (0, 0)),
          pl.BlockSpec((B, D), lambda i: (0, 0)),
          pl.BlockSpec((B, D), lambda i: (0, 0)),
      ],
      out_specs=(
          pl.BlockSpec((512, 3 * D), lambda i: (i, 0)),
          pl.BlockSpec((512, 3 * D), lambda i: (i, 0)),
      ),
      out_shape=(
          jax.ShapeDtypeStruct((NPAD, 3 * D), jnp.float32),
          jax.ShapeDtypeStruct((NPAD, 3 * D), jnp.float32),
      ),
  )(node_pad, M, U, V)


def _t2_body(rel_ref, m11_ref, z_ref):
  relb = rel_ref[...]
  m11 = m11_ref[...]
  t = jax.lax.dot_general(relb, m11, (((2,), (1,)), ((), ())),
                          precision=jax.lax.Precision.HIGHEST,
                          preferred_element_type=jnp.float32)
  z_ref[...] = jnp.sum(relb * t, axis=2)


def _quad_form(rel3, M11):
  grid = 25
  return pl.pallas_call(
      _t2_body,
      grid=(grid,),
      in_specs=[
          pl.BlockSpec((40, 160, D), lambda i: (i, 0, 0)),
          pl.BlockSpec((D, D), lambda i: (0, 0)),
      ],
      out_specs=pl.BlockSpec((40, 160), lambda i: (i, 0)),
      out_shape=jax.ShapeDtypeStruct((1000, 160), jnp.float32),
  )(rel3, M11)


def _t5_body(ag_ref, sc_ref, ms_ref, nb_ref, w_ref, b_ref, out_ref, score_ref):
  agg = ag_ref[0] + ag_ref[1]
  mask = ms_ref[...]
  upd = agg + mask * nb_ref[...]
  uu = jax.lax.bitcast_convert_type(upd, jnp.int32)
  ur = (uu + 0x7FFF + ((uu >> 16) & 1)) & jnp.int32(-65536)
  updr = jax.lax.bitcast_convert_type(ur, jnp.float32)
  out = _hdot(updr, w_ref[...].T) + b_ref[...]
  out_ref[...] = jnp.where(out >= 0.0, out, NEG_SLOPE * out)
  score_ref[...] = sc_ref[0] + sc_ref[1]


def _finalize(agg2, score2r, ms2d, node_pad, W_lin, b_lin2):
  grid = NPAD // 512
  return pl.pallas_call(
      _t5_body,
      grid=(grid,),
      in_specs=[
          pl.BlockSpec((2, 512, D), lambda i: (0, i, 0)),
          pl.BlockSpec((2, 1, 1, 512), lambda i: (0, i, 0, 0)),
          pl.BlockSpec((512, 1), lambda i: (i, 0)),
          pl.BlockSpec((512, D), lambda i: (i, 0)),
          pl.BlockSpec((D, D), lambda i: (0, 0)),
          pl.BlockSpec((1, D), lambda i: (0, 0)),
      ],
      out_specs=(
          pl.BlockSpec((512, D), lambda i: (i, 0)),
          pl.BlockSpec((1, 1, 512), lambda i: (i, 0, 0)),
      ),
      out_shape=(
          jax.ShapeDtypeStruct((NPAD, D), jnp.float32),
          jax.ShapeDtypeStruct((NPAD // 512, 1, 512), jnp.float32),
      ),
  )(agg2, score2r, ms2d, node_pad, W_lin, b_lin2)


_MESH = plsc.VectorSubcoreMesh(core_axis_name="c", subcore_axis_name="s")


def _wid():
  return lax.axis_index("s") * NC + lax.axis_index("c")


def _iota16():
  return lax.iota(jnp.int32, 16)


def _splat(x):
  return jnp.broadcast_to(x, (16,))


def _p1_body(tsrc, tdst, qtab, relh, esh, edh, qih, zh,
             logits_out, lmax_out,
             bufS0, bufS1, bufD0, bufD1, relb0, relb1, eid0, eid1,
             esv, edv, qiv, zv, bufQ, lgbuf, kscr, vscr, lmax,
             smS0, smS1, smD0, smD1, smR0, smR1):
  wid = _wid()
  ebase = wid * EW
  bufS_ = (bufS0, bufS1)
  bufD_ = (bufD0, bufD1)
  relb_ = (relb0, relb1)
  eid_ = (eid0, eid1)
  smS_ = (smS0, smS1)
  smD_ = (smD0, smD1)
  smR_ = (smR0, smR1)
  ROUNDS = EW // C1

  neg = jnp.full((16,), -1e30, jnp.float32)

  def init_body(k, _):
    lmax[pl.ds(k * 16, 16)] = neg
    return 0

  lax.fori_loop(0, NPAD // 16, init_body, 0)

  pltpu.sync_copy(qtab, bufQ)
  pltpu.sync_copy(esh.at[pl.ds(ebase, EW)], esv)
  pltpu.sync_copy(edh.at[pl.ds(ebase, EW)], edv)
  pltpu.sync_copy(qih.at[pl.ds(ebase, EW)], qiv)
  pltpu.sync_copy(zh.at[pl.ds(ebase, EW)], zv)

  iota = _iota16()
  emax = _splat(E - 1)

  def issue(r, b):
    lb = r * C1
    for g in range(C1 // 16):
      eid_[b][pl.ds(g * 16, 16)] = jnp.minimum(
          _splat(ebase + lb + g * 16) + iota, emax)
    pltpu.async_copy(tsrc.at[esv.at[pl.ds(lb, C1)]], bufS_[b], smS_[b])
    pltpu.async_copy(tdst.at[edv.at[pl.ds(lb, C1)]], bufD_[b], smD_[b])
    pltpu.async_copy(relh.at[eid_[b]], relb_[b], smR_[b])

  def drain(b):
    pltpu.make_async_copy(tsrc.at[pl.ds(0, C1)], bufS_[b], smS_[b]).wait()
    pltpu.make_async_copy(tdst.at[pl.ds(0, C1)], bufD_[b], smD_[b]).wait()
    pltpu.make_async_copy(relh.at[pl.ds(0, C1)], relb_[b], smR_[b]).wait()

  def compute(r, b):
    lb = r * C1
    bufS = bufS_[b]
    bufD = bufD_[b]
    relbuf = relb_[b]
    for g in range(C1 // 16):
      lane = iota + (g * 16)
      s16 = esv[pl.ds(lb + g * 16, 16)]
      iq16 = qiv[pl.ds(lb + g * 16, 16)]
      z16 = zv[pl.ds(lb + g * 16, 16)]
      c16 = plsc.load_gather(bufQ, [iq16, _splat(D)])
      x16 = plsc.load_gather(bufS, [lane, _splat(2 * D) + iq16])
      y16 = plsc.load_gather(bufD, [lane, _splat(2 * D) + iq16])
      acc0 = z16 + c16 + x16 + y16

      def dot_body(j, acc):
        for jj in range(4):
          jb = _splat(j * 4 + jj)
          r16 = plsc.load_gather(relbuf, [lane, jb])
          gp16 = plsc.load_gather(bufS, [lane, jb])
          p16 = plsc.load_gather(bufS, [lane, _splat(D) + jb])
          g16 = plsc.load_gather(bufD, [lane, jb])
          nr16 = plsc.load_gather(bufD, [lane, _splat(D) + jb])
          h16 = plsc.load_gather(bufQ, [iq16, jb])
          acc = acc + r16 * (gp16 + g16 + h16) + p16 * nr16
        return acc

      acc = lax.fori_loop(0, D // 4, dot_body, acc0)
      lgbuf[pl.ds(g * 16, 16)] = acc

      key, val = plsc.sort_key_val(s16, acc)
      for sh in (1, 2, 4, 8):
        kscr[...] = key
        vscr[...] = val
        back = jnp.maximum(iota - sh, 0)
        kb = plsc.load_gather(kscr, [back])
        vb = plsc.load_gather(vscr, [back])
        same = jnp.logical_and(kb == key, iota >= sh)
        val = jnp.where(same, jnp.maximum(val, vb), val)
      kscr[...] = key
      nxt = jnp.minimum(iota + 1, 15)
      kn = plsc.load_gather(kscr, [nxt])
      is_last = jnp.logical_or(kn != key, iota == 15)
      cur = plsc.load_gather(lmax, [key])
      plsc.store_scatter(lmax, [key], jnp.maximum(cur, val), mask=is_last)

    pltpu.sync_copy(lgbuf, logits_out.at[pl.ds(ebase + lb, C1)])

  issue(jnp.int32(0), 0)
  last = jnp.int32(ROUNDS - 1)

  def pair_body(h, _):
    r0 = h * 2
    drain(0)
    issue(r0 + 1, 1)
    compute(r0, 0)
    drain(1)
    issue(jnp.minimum(r0 + 2, last), 0)
    compute(r0 + 1, 1)
    return 0

  lax.fori_loop(0, ROUNDS // 2, pair_body, 0)
  drain(0)
  pltpu.sync_copy(lmax, lmax_out.at[wid])


def _run_p1(tsrc, tdst, qtab, relh, esh, edh, qih, zh):
  return pl.kernel(
      _p1_body,
      out_type=(
          jax.ShapeDtypeStruct((EPAD,), jnp.float32),
          jax.ShapeDtypeStruct((NW, NPAD), jnp.float32),
      ),
      mesh=_MESH,
      compiler_params=pltpu.CompilerParams(use_tc_tiling_on_sc=False, needs_layout_passes=False),
      scratch_types=[
          pltpu.VMEM((C1, 3 * D), jnp.float32),
          pltpu.VMEM((C1, 3 * D), jnp.float32),
          pltpu.VMEM((C1, 3 * D), jnp.float32),
          pltpu.VMEM((C1, 3 * D), jnp.float32),
          pltpu.VMEM((C1, D), jnp.float32),
          pltpu.VMEM((C1, D), jnp.float32),
          pltpu.VMEM((C1,), jnp.int32),
          pltpu.VMEM((C1,), jnp.int32),
          pltpu.VMEM((EW,), jnp.int32),
          pltpu.VMEM((EW,), jnp.int32),
          pltpu.VMEM((EW,), jnp.int32),
          pltpu.VMEM((EW,), jnp.float32),
          pltpu.VMEM((B, QCOLS), jnp.float32),
          pltpu.VMEM((C1,), jnp.float32),
          pltpu.VMEM((16,), jnp.int32),
          pltpu.VMEM((16,), jnp.float32),
          pltpu.VMEM((NPAD,), jnp.float32),
          pltpu.SemaphoreType.DMA,
          pltpu.SemaphoreType.DMA,
          pltpu.SemaphoreType.DMA,
          pltpu.SemaphoreType.DMA,
          pltpu.SemaphoreType.DMA,
          pltpu.SemaphoreType.DMA,
      ],
  )(tsrc, tdst, qtab, relh, esh, edh, qih, zh)


def _p3_body(lmaxh, logitsh, esh, exh, denomh,
             gmax, mbuf, lgbuf, exbuf, isrcv, zv, denom_sp):
  cid = lax.axis_index("c")
  sid = lax.axis_index("s")
  wid = sid * NC + cid
  ebase = wid * EW

  zero = jnp.zeros((16,), jnp.float32)

  def z_body(k, _):
    zv[pl.ds(k * 16, 16)] = zero
    return 0

  lax.fori_loop(0, (NPAD // NS) // 16, z_body, 0)
  pltpu.sync_copy(zv, denom_sp.at[pl.ds(sid * (NPAD // NS), NPAD // NS)])

  def cmb_outer(kk, _):
    pltpu.sync_copy(lmaxh.at[:, pl.ds(kk * 2048, 2048)], mbuf)

    def cmb_inner(j, _):
      m = mbuf[0, pl.ds(j * 16, 16)]
      for w in range(1, NW):
        m = jnp.maximum(m, mbuf[w, pl.ds(j * 16, 16)])
      gmax[pl.ds(kk * 2048 + j * 16, 16)] = m
      return 0

    lax.fori_loop(0, 2048 // 16, cmb_inner, 0)
    return 0

  lax.fori_loop(0, NPAD // 2048, cmb_outer, 0)
  plsc.subcore_barrier()

  def round_body(r, _):
    base = ebase + r * C3
    pltpu.sync_copy(logitsh.at[pl.ds(base, C3)], lgbuf)
    pltpu.sync_copy(esh.at[pl.ds(base, C3)], isrcv)
    for g in range(C3 // 16):
      lg16 = lgbuf[pl.ds(g * 16, 16)]
      s16 = isrcv[pl.ds(g * 16, 16)]
      gm16 = plsc.load_gather(gmax, [s16])
      exbuf[pl.ds(g * 16, 16)] = jnp.exp(lg16 - gm16)
    pltpu.sync_copy(exbuf, exh.at[pl.ds(base, C3)])
    pltpu.sync_copy(exbuf, denom_sp.at[isrcv], add=True)
    return 0

  lax.fori_loop(0, EW // C3, round_body, 0)
  plsc.subcore_barrier()
  sl = pl.ds(sid * (NPAD // NS), NPAD // NS)
  pltpu.sync_copy(denom_sp.at[sl], denomh.at[cid, sl])


def _run_p3(lmaxh, logitsh, esh):
  return pl.kernel(
      _p3_body,
      out_type=(
          jax.ShapeDtypeStruct((EPAD,), jnp.float32),
          jax.ShapeDtypeStruct((NC, NPAD), jnp.float32),
      ),
      mesh=_MESH,
      compiler_params=pltpu.CompilerParams(use_tc_tiling_on_sc=False, needs_layout_passes=False),
      scratch_types=[
          pltpu.VMEM((NPAD,), jnp.float32),
          pltpu.VMEM((NW, 2048), jnp.float32),
          pltpu.VMEM((C3,), jnp.float32),
          pltpu.VMEM((C3,), jnp.float32),
          pltpu.VMEM((C3,), jnp.int32),
          pltpu.VMEM((NPAD // NS,), jnp.float32),
          pltpu.VMEM_SHARED((NPAD,), jnp.float32),
      ],
  )(lmaxh, logitsh, esh)


def _p4_body(exh, esh, edh, denomh, vnsh, nodeh,
             scoreh, aggh, msh,
             denv, vnsv, dbuf, exbuf0, exbuf1, sbuf0, sbuf1,
             isrcv0, isrcv1, idstv0, idstv1, ndbuf0, ndbuf1,
             zrows, msv, score_sp, agg_sp, semN0, semN1):
  cid = lax.axis_index("c")
  sid = lax.axis_index("s")
  wid = sid * NC + cid
  ebase = wid * EW
  rows = NPAD // NS

  zero = jnp.zeros((16,), jnp.float32)

  def zr_body(k, _):
    for jj in range(D // 16):
      zrows[k, pl.ds(jj * 16, 16)] = zero
    return 0

  lax.fori_loop(0, 40, zr_body, 0)

  def zv_body(k, _):
    msv[pl.ds(k * 16, 16)] = zero
    return 0

  lax.fori_loop(0, rows // 16, zv_body, 0)
  pltpu.sync_copy(msv, score_sp.at[pl.ds(sid * rows, rows)])
  for t in range(16):
    pltpu.sync_copy(zrows, agg_sp.at[pl.ds(sid * rows + t * 40, 40), :])

  def den_outer(kk, _):
    pltpu.sync_copy(denomh.at[:, pl.ds(kk * 2048, 2048)], dbuf)

    def den_inner(j, _):
      denv[pl.ds(kk * 2048 + j * 16, 16)] = (
          dbuf[0, pl.ds(j * 16, 16)] + dbuf[1, pl.ds(j * 16, 16)])
      return 0

    lax.fori_loop(0, 2048 // 16, den_inner, 0)
    return 0

  lax.fori_loop(0, NPAD // 2048, den_outer, 0)
  pltpu.sync_copy(vnsh, vnsv)
  plsc.subcore_barrier()

  iota = _iota16()
  exbuf_ = (exbuf0, exbuf1)
  sbuf_ = (sbuf0, sbuf1)
  isrcv_ = (isrcv0, isrcv1)
  idstv_ = (idstv0, idstv1)
  ndbuf_ = (ndbuf0, ndbuf1)
  semN_ = (semN0, semN1)
  ROUNDS4 = EW // C4

  def issue(r, b):
    base = ebase + r * C4
    pltpu.sync_copy(exh.at[pl.ds(base, C4)], exbuf_[b])
    pltpu.sync_copy(esh.at[pl.ds(base, C4)], isrcv_[b])
    pltpu.sync_copy(edh.at[pl.ds(base, C4)], idstv_[b])
    pltpu.async_copy(nodeh.at[idstv_[b]], ndbuf_[b], semN_[b])

  def drain(b):
    pltpu.make_async_copy(nodeh.at[pl.ds(0, C4)], ndbuf_[b], semN_[b]).wait()

  def compute(r, b):
    exbuf = exbuf_[b]
    sbuf = sbuf_[b]
    isrcv = isrcv_[b]
    idstv = idstv_[b]
    ndbuf = ndbuf_[b]
    sms = []
    for g in range(C4 // 16):
      e16 = exbuf[pl.ds(g * 16, 16)]
      s16 = isrcv[pl.ds(g * 16, 16)]
      den16 = plsc.load_gather(denv, [s16])
      sm16 = e16 / (den16 + 1e-16)
      vn16 = plsc.load_gather(vnsv, [s16])
      sbuf[pl.ds(g * 16, 16)] = sm16 * vn16
      sms.append(sm16)

    def sc_body(j, _):
      jb = _splat(j)
      for g in range(C4 // 16):
        lane = iota + (g * 16)
        val = plsc.load_gather(ndbuf, [lane, jb]) * sms[g]
        plsc.store_scatter(ndbuf, [lane, jb], val)
      return 0

    lax.fori_loop(0, D, sc_body, 0)
    pltpu.sync_copy(ndbuf, agg_sp.at[isrcv], add=True)
    pltpu.sync_copy(sbuf, score_sp.at[idstv], add=True)

  issue(jnp.int32(0), 0)
  last4 = jnp.int32(ROUNDS4 - 1)

  def pair_body(h, _):
    r0 = h * 2
    drain(0)
    issue(r0 + 1, 1)
    compute(r0, 0)
    drain(1)
    issue(jnp.minimum(r0 + 2, last4), 0)
    compute(r0 + 1, 1)
    return 0

  lax.fori_loop(0, ROUNDS4 // 2, pair_body, 0)
  drain(0)

  def ms_body(k, _):
    d16 = denv[pl.ds(sid * rows + k * 16, 16)]
    msv[pl.ds(k * 16, 16)] = jnp.where(d16 > 0.0, 0.0, 1.0)
    return 0

  lax.fori_loop(0, rows // 16, ms_body, 0)

  plsc.subcore_barrier()
  sl = pl.ds(sid * rows, rows)
  pltpu.sync_copy(score_sp.at[sl], scoreh.at[cid, sl])
  pltpu.sync_copy(agg_sp.at[sl, :], aggh.at[cid, sl, :])

  @pl.when(cid == 0)
  def _():
    pltpu.sync_copy(msv, msh.at[sl])


def _run_p4(exh, esh, edh, denomh, vnsh, nodeh):
  return pl.kernel(
      _p4_body,
      out_type=(
          jax.ShapeDtypeStruct((NC, NPAD), jnp.float32),
          jax.ShapeDtypeStruct((NC, NPAD, D), jnp.float32),
          jax.ShapeDtypeStruct((NPAD,), jnp.float32),
      ),
      mesh=_MESH,
      compiler_params=pltpu.CompilerParams(use_tc_tiling_on_sc=False, needs_layout_passes=False),
      scratch_types=[
          pltpu.VMEM((NPAD,), jnp.float32),
          pltpu.VMEM((NPAD,), jnp.float32),
          pltpu.VMEM((NC, 2048), jnp.float32),
          pltpu.VMEM((C4,), jnp.float32),
          pltpu.VMEM((C4,), jnp.float32),
          pltpu.VMEM((C4,), jnp.float32),
          pltpu.VMEM((C4,), jnp.float32),
          pltpu.VMEM((C4,), jnp.int32),
          pltpu.VMEM((C4,), jnp.int32),
          pltpu.VMEM((C4,), jnp.int32),
          pltpu.VMEM((C4,), jnp.int32),
          pltpu.VMEM((C4, D), jnp.float32),
          pltpu.VMEM((C4, D), jnp.float32),
          pltpu.VMEM((40, D), jnp.float32),
          pltpu.VMEM((NPAD // NS,), jnp.float32),
          pltpu.VMEM_SHARED((NPAD,), jnp.float32),
          pltpu.VMEM_SHARED((NPAD, D), jnp.float32),
          pltpu.SemaphoreType.DMA,
          pltpu.SemaphoreType.DMA,
      ],
  )(exh, esh, edh, denomh, vnsh, nodeh)


@jax.jit
def kernel(node_repr, rel_emb, query_src_ts_emb, query_rel_emb,
           visited_node_score, Wq, Wk, W_lin, b_lin,
           edge_src, edge_dst, query_idx):
  def _r(x):
    u = jax.lax.bitcast_convert_type(x, jnp.int32)
    r = (u + 0x7FFF + ((u >> 16) & 1)) & jnp.int32(-65536)
    return jax.lax.bitcast_convert_type(r, jnp.float32)

  node_pad = jnp.concatenate(
      [node_repr, jnp.zeros((NPAD - N, D), jnp.float32)], axis=0)
  node_pad_r = _r(node_pad)
  rel_r = _r(rel_emb)
  vns_pad = jnp.concatenate(
      [visited_node_score, jnp.zeros((NPAD - N,), jnp.float32)])
  pad_i = jnp.full((EPAD - E,), N, jnp.int32)
  es_pad = jnp.concatenate([edge_src, pad_i])
  ed_pad = jnp.concatenate([edge_dst, pad_i])
  qi_pad = jnp.concatenate([query_idx, jnp.zeros((EPAD - E,), jnp.int32)])

  M, qtab, U, V = _tables_small(_r(Wq), _r(Wk), _r(query_src_ts_emb),
                                _r(query_rel_emb))
  tsrc, tdst = _tables_node(node_pad_r, M, U, V)
  z2d = _quad_form(rel_r.reshape(1000, 160, D), M[D:2 * D, D:2 * D])
  z_pad = jnp.concatenate(
      [z2d.reshape(E), jnp.zeros((EPAD - E,), jnp.float32)])

  logits, lmaxh = _run_p1(tsrc, tdst, qtab, rel_r,
                          es_pad, ed_pad, qi_pad, z_pad)
  exh, denomh = _run_p3(lmaxh, logits, es_pad)
  scoreh, aggh, msh = _run_p4(exh, es_pad, ed_pad, denomh, vns_pad, node_pad)

  out_repr_pad, score2d = _finalize(
      aggh, scoreh.reshape(NC, NPAD // 512, 1, 512), msh.reshape(NPAD, 1),
      node_pad, _r(W_lin), b_lin.reshape(1, D))
  return score2d.reshape(NPAD)[:N], out_repr_pad[:N]
